# Initial kernel scaffold; baseline (speedup 1.0000x reference)
import jax
import jax.numpy as jnp
from jax.experimental import pallas as pl


def kernel(drug_idx, adr_idx, drug_embeddings, adr_embeddings, bias_d, bias_a, L_w, L_b):
    raise NotImplementedError("write your pallas kernel here")


# speedup vs baseline: 1.7489x; 1.7489x over previous
import jax
import jax.numpy as jnp
from jax import lax
from jax.experimental import pallas as pl
from jax.experimental.pallas import tpu as pltpu
from jax.experimental.pallas import tpu_sc as plsc

N_CORES = 2
N_SUBCORES = 16
NW = N_CORES * N_SUBCORES

B = 16384
FPT_DIM = 256
DIM = 64
B_PER_W = B // NW
CHUNK = 128
N_CHUNKS = B_PER_W // CHUNK


def _sc_gather_body(didx_hbm, aidx_hbm, demb_hbm, aemb_hbm, bd_hbm, ba_hbm,
                    drugs_out, adrs_out, bd_out, ba_out,
                    didx_v, aidx_v, dbuf0, dbuf1, abuf, bdbuf, babuf,
                    sem0, sem1, sem2):
  wid = lax.axis_index("s") * N_CORES + lax.axis_index("c")
  base = wid * B_PER_W

  for k in range(N_CHUNKS):
    pltpu.sync_copy(didx_hbm.at[pl.ds(base + k * CHUNK, CHUNK)], didx_v.at[k])
    pltpu.sync_copy(aidx_hbm.at[pl.ds(base + k * CHUNK, CHUNK)], aidx_v.at[k])

  small = []
  for k in range(N_CHUNKS):
    small.append(pltpu.async_copy(
        aemb_hbm.at[aidx_v.at[k]], abuf.at[pl.ds(k * CHUNK, CHUNK)], sem1))
  for k in range(N_CHUNKS):
    small.append(pltpu.async_copy(
        bd_hbm.at[didx_v.at[k]], bdbuf.at[pl.ds(k * CHUNK, CHUNK)], sem2))
    small.append(pltpu.async_copy(
        ba_hbm.at[aidx_v.at[k]], babuf.at[pl.ds(k * CHUNK, CHUNK)], sem2))

  bufs = (dbuf0, dbuf1)
  cps = [None] * N_CHUNKS
  cps[0] = pltpu.async_copy(demb_hbm.at[didx_v.at[0]], bufs[0], sem0)
  for k in range(N_CHUNKS):
    if k + 1 < N_CHUNKS:
      cps[k + 1] = pltpu.async_copy(
          demb_hbm.at[didx_v.at[k + 1]], bufs[(k + 1) % 2], sem0)
    cps[k].wait()
    pltpu.sync_copy(bufs[k % 2],
                    drugs_out.at[pl.ds(base + k * CHUNK, CHUNK)])

  for cp in small:
    cp.wait()
  pltpu.sync_copy(abuf, adrs_out.at[pl.ds(base, B_PER_W)])
  pltpu.sync_copy(bdbuf, bd_out.at[pl.ds(base, B_PER_W)])
  pltpu.sync_copy(babuf, ba_out.at[pl.ds(base, B_PER_W)])


def _sc_gather(drug_idx, adr_idx, drug_embeddings, adr_embeddings, bias_d,
               bias_a):
  mesh = plsc.VectorSubcoreMesh(core_axis_name="c", subcore_axis_name="s")
  out_type = (
      jax.ShapeDtypeStruct((B, FPT_DIM), jnp.float32),
      jax.ShapeDtypeStruct((B, DIM), jnp.float32),
      jax.ShapeDtypeStruct((B,), jnp.float32),
      jax.ShapeDtypeStruct((B,), jnp.float32),
  )
  scratch = [
      pltpu.VMEM((N_CHUNKS, CHUNK), jnp.int32),
      pltpu.VMEM((N_CHUNKS, CHUNK), jnp.int32),
      pltpu.VMEM((CHUNK, FPT_DIM), jnp.float32),
      pltpu.VMEM((CHUNK, FPT_DIM), jnp.float32),
      pltpu.VMEM((B_PER_W, DIM), jnp.float32),
      pltpu.VMEM((B_PER_W,), jnp.float32),
      pltpu.VMEM((B_PER_W,), jnp.float32),
      pltpu.SemaphoreType.DMA,
      pltpu.SemaphoreType.DMA,
      pltpu.SemaphoreType.DMA,
  ]
  fn = pl.kernel(_sc_gather_body, out_type=out_type, mesh=mesh,
                 scratch_types=scratch,
                 compiler_params=pltpu.CompilerParams(use_tc_tiling_on_sc=False))
  return fn(drug_idx, adr_idx, drug_embeddings, adr_embeddings,
            bias_d.reshape(-1), bias_a.reshape(-1))


def _tc_score_body(drugs_ref, adrs_ref, bd_ref, ba_ref, lw_ref, lb_ref,
                   out_ref):
  proj = lax.dot_general(drugs_ref[...], lw_ref[...],
                         (((1,), (1,)), ((), ())),
                         preferred_element_type=jnp.float32)
  proj = proj + lb_ref[...]
  s = jnp.sum(proj * adrs_ref[...], axis=1)
  out_ref[...] = s + bd_ref[...] + ba_ref[...]


def _tc_score(drugs_g, adrs_g, bd_g, ba_g, L_w, L_b):
  blk = 2048
  grid = (B // blk,)
  return pl.pallas_call(
      _tc_score_body,
      grid=grid,
      in_specs=[
          pl.BlockSpec((blk, FPT_DIM), lambda i: (i, 0)),
          pl.BlockSpec((blk, DIM), lambda i: (i, 0)),
          pl.BlockSpec((blk,), lambda i: (i,)),
          pl.BlockSpec((blk,), lambda i: (i,)),
          pl.BlockSpec((DIM, FPT_DIM), lambda i: (0, 0)),
          pl.BlockSpec((1, DIM), lambda i: (0, 0)),
      ],
      out_specs=pl.BlockSpec((blk,), lambda i: (i,)),
      out_shape=jax.ShapeDtypeStruct((B,), jnp.float32),
  )(drugs_g, adrs_g, bd_g, ba_g, L_w, L_b.reshape(1, DIM))


def kernel(drug_idx, adr_idx, drug_embeddings, adr_embeddings, bias_d, bias_a,
           L_w, L_b):
  drug_idx = drug_idx.astype(jnp.int32)
  adr_idx = adr_idx.astype(jnp.int32)
  drugs_g, adrs_g, bd_g, ba_g = _sc_gather(
      drug_idx, adr_idx, drug_embeddings, adr_embeddings, bias_d, bias_a)
  return _tc_score(drugs_g, adrs_g, bd_g, ba_g, L_w, L_b)


# speedup vs baseline: 2.7578x; 1.5769x over previous
import jax
import jax.numpy as jnp
from jax import lax
from jax.experimental import pallas as pl
from jax.experimental.pallas import tpu as pltpu
from jax.experimental.pallas import tpu_sc as plsc

N_CORES = 2
N_SUBCORES = 16
NW = N_CORES * N_SUBCORES

B = 16384
FPT_DIM = 256
DIM = 64
B_PER_W = B // NW
CHUNK = 128
N_CHUNKS = B_PER_W // CHUNK


def _sc_drug_body(didx_hbm, demb_hbm, drugs_out, didx_v, dbuf0, dbuf1, sem0):
  wid = lax.axis_index("s") * N_CORES + lax.axis_index("c")
  base = wid * B_PER_W

  for k in range(N_CHUNKS):
    pltpu.sync_copy(didx_hbm.at[pl.ds(base + k * CHUNK, CHUNK)], didx_v.at[k])

  bufs = (dbuf0, dbuf1)
  cps = [None] * N_CHUNKS
  cps[0] = pltpu.async_copy(demb_hbm.at[didx_v.at[0]], bufs[0], sem0)
  for k in range(N_CHUNKS):
    if k + 1 < N_CHUNKS:
      cps[k + 1] = pltpu.async_copy(
          demb_hbm.at[didx_v.at[k + 1]], bufs[(k + 1) % 2], sem0)
    cps[k].wait()
    pltpu.sync_copy(bufs[k % 2],
                    drugs_out.at[pl.ds(base + k * CHUNK, CHUNK)])


def _sc_adr_body(didx_hbm, aidx_hbm, aemb_hbm, bd_hbm, ba_hbm,
                 adrs_out, bd_out, ba_out,
                 didx_v, aidx_v, abuf, bdbuf, babuf, sem1, sem2):
  wid = lax.axis_index("s") * N_CORES + lax.axis_index("c")
  base = wid * B_PER_W

  for k in range(N_CHUNKS):
    pltpu.sync_copy(didx_hbm.at[pl.ds(base + k * CHUNK, CHUNK)], didx_v.at[k])
    pltpu.sync_copy(aidx_hbm.at[pl.ds(base + k * CHUNK, CHUNK)], aidx_v.at[k])

  cps = []
  for k in range(N_CHUNKS):
    cps.append(pltpu.async_copy(
        aemb_hbm.at[aidx_v.at[k]], abuf.at[pl.ds(k * CHUNK, CHUNK)], sem1))
  for k in range(N_CHUNKS):
    cps.append(pltpu.async_copy(
        bd_hbm.at[didx_v.at[k]], bdbuf.at[pl.ds(k * CHUNK, CHUNK)], sem2))
    cps.append(pltpu.async_copy(
        ba_hbm.at[aidx_v.at[k]], babuf.at[pl.ds(k * CHUNK, CHUNK)], sem2))
  for cp in cps:
    cp.wait()
  pltpu.sync_copy(abuf, adrs_out.at[pl.ds(base, B_PER_W)])
  pltpu.sync_copy(bdbuf, bd_out.at[pl.ds(base, B_PER_W)])
  pltpu.sync_copy(babuf, ba_out.at[pl.ds(base, B_PER_W)])


def _sc_gather(drug_idx, adr_idx, drug_embeddings, adr_embeddings, bias_d,
               bias_a):
  mesh = plsc.VectorSubcoreMesh(core_axis_name="c", subcore_axis_name="s")

  drug_fn = pl.kernel(
      _sc_drug_body,
      out_type=jax.ShapeDtypeStruct((B, FPT_DIM), jnp.float32),
      mesh=mesh,
      scratch_types=[
          pltpu.VMEM((8, CHUNK), jnp.int32),
          pltpu.VMEM((CHUNK, FPT_DIM), jnp.float32),
          pltpu.VMEM((CHUNK, FPT_DIM), jnp.float32),
          pltpu.SemaphoreType.DMA,
      ],
      compiler_params=pltpu.CompilerParams(use_tc_tiling_on_sc=True))
  drugs_g = drug_fn(drug_idx, drug_embeddings)

  adr_fn = pl.kernel(
      _sc_adr_body,
      out_type=(
          jax.ShapeDtypeStruct((B, DIM), jnp.float32),
          jax.ShapeDtypeStruct((B,), jnp.float32),
          jax.ShapeDtypeStruct((B,), jnp.float32),
      ),
      mesh=mesh,
      scratch_types=[
          pltpu.VMEM((N_CHUNKS, CHUNK), jnp.int32),
          pltpu.VMEM((N_CHUNKS, CHUNK), jnp.int32),
          pltpu.VMEM((B_PER_W, DIM), jnp.float32),
          pltpu.VMEM((B_PER_W,), jnp.float32),
          pltpu.VMEM((B_PER_W,), jnp.float32),
          pltpu.SemaphoreType.DMA,
          pltpu.SemaphoreType.DMA,
      ],
      compiler_params=pltpu.CompilerParams(use_tc_tiling_on_sc=False))
  adrs_g, bd_g, ba_g = adr_fn(drug_idx, adr_idx, adr_embeddings,
                              bias_d.reshape(-1), bias_a.reshape(-1))
  return drugs_g, adrs_g, bd_g, ba_g


def _tc_score_body(drugs_ref, adrs_ref, bd_ref, ba_ref, lw_ref, lb_ref,
                   out_ref):
  proj = lax.dot_general(drugs_ref[...], lw_ref[...],
                         (((1,), (1,)), ((), ())),
                         preferred_element_type=jnp.float32)
  proj = proj + lb_ref[...]
  s = jnp.sum(proj * adrs_ref[...], axis=1)
  out_ref[...] = s + bd_ref[...] + ba_ref[...]


def _tc_score(drugs_g, adrs_g, bd_g, ba_g, L_w, L_b):
  blk = 2048
  grid = (B // blk,)
  return pl.pallas_call(
      _tc_score_body,
      grid=grid,
      in_specs=[
          pl.BlockSpec((blk, FPT_DIM), lambda i: (i, 0)),
          pl.BlockSpec((blk, DIM), lambda i: (i, 0)),
          pl.BlockSpec((blk,), lambda i: (i,)),
          pl.BlockSpec((blk,), lambda i: (i,)),
          pl.BlockSpec((DIM, FPT_DIM), lambda i: (0, 0)),
          pl.BlockSpec((1, DIM), lambda i: (0, 0)),
      ],
      out_specs=pl.BlockSpec((blk,), lambda i: (i,)),
      out_shape=jax.ShapeDtypeStruct((B,), jnp.float32),
  )(drugs_g, adrs_g, bd_g, ba_g, L_w, L_b.reshape(1, DIM))


def kernel(drug_idx, adr_idx, drug_embeddings, adr_embeddings, bias_d, bias_a,
           L_w, L_b):
  drug_idx = drug_idx.astype(jnp.int32)
  adr_idx = adr_idx.astype(jnp.int32)
  drugs_g, adrs_g, bd_g, ba_g = _sc_gather(
      drug_idx, adr_idx, drug_embeddings, adr_embeddings, bias_d, bias_a)
  return _tc_score(drugs_g, adrs_g, bd_g, ba_g, L_w, L_b)
